# Initial kernel scaffold; baseline (speedup 1.0000x reference)
import jax
import jax.numpy as jnp
from jax.experimental import pallas as pl


def kernel(x, s_num):
    raise NotImplementedError("write your pallas kernel here")


# speedup vs baseline: 3.5076x; 3.5076x over previous
import jax
import jax.numpy as jnp
from jax import lax
from jax.experimental import pallas as pl

B = 4
C = 5
P = 8192
S = 2048
K = 16
S_BLK = 128
NSB = S // S_BLK


def _sample_perm(s_num):
    rk = jax.random.key(1)
    rand_num = jnp.abs(jax.random.uniform(rk, (B, P), dtype=jnp.float32))
    sorted_idx = jnp.argsort(rand_num, axis=1)
    return jax.lax.dynamic_slice(sorted_idx, (0, s_num - S), (B, S))


def _body(x_ref, perm_ref, out_ref):
    x5 = x_ref[0]
    x3 = x5[:3, :]
    perm = perm_ref[0, 0]

    iota_p = lax.broadcasted_iota(jnp.int32, (S_BLK, P), 1)

    ohq = (iota_p == perm[:, None]).astype(jnp.float32)
    q = lax.dot_general(ohq, x3, (((1,), (1,)), ((), ())),
                        precision=lax.Precision.HIGHEST)

    xx = jnp.sum(x3 * x3, axis=0)
    xx2 = jnp.sum(q * q, axis=1)
    inner = -2.0 * lax.dot_general(q, x3, (((1,), (0,)), ((), ())))
    z = ((-xx)[None, :] - inner) - xx2[:, None]

    for k in range(K):
        m = jnp.max(z, axis=1)
        cand = jnp.where(z == m[:, None], iota_p, jnp.int32(P))
        pidx = jnp.min(cand, axis=1)
        oh = (iota_p == pidx[:, None]).astype(jnp.float32)
        f_t = lax.dot_general(x5, oh, (((1,), (1,)), ((), ())),
                              precision=lax.Precision.HIGHEST)
        out_ref[0, :, k, :] = f_t
        z = jnp.where(iota_p == pidx[:, None], -jnp.inf, z)


def kernel(x, s_num):
    perm = _sample_perm(s_num).astype(jnp.int32)
    perm3 = perm.reshape(B * NSB, 1, S_BLK)
    out = pl.pallas_call(
        _body,
        grid=(B, NSB),
        in_specs=[
            pl.BlockSpec((1, C, P), lambda b, s: (b, 0, 0)),
            pl.BlockSpec((1, 1, S_BLK), lambda b, s: (b * NSB + s, 0, 0)),
        ],
        out_specs=pl.BlockSpec((1, C, K, S_BLK), lambda b, s: (b, 0, 0, s)),
        out_shape=jax.ShapeDtypeStruct((B, C, K, S), jnp.float32),
    )(x, perm3)
    return out.reshape(B, C, S, K)


# speedup vs baseline: 4.1974x; 1.1966x over previous
import functools

import jax
import jax.numpy as jnp
from jax import lax
from jax.experimental import pallas as pl
from jax.experimental.pallas import tpu as pltpu
from jax.experimental.pallas import tpu_sc as plsc

B = 4
C = 5
P = 8192
S = 2048
K = 16
S_BLK = 128
NSB = S // S_BLK
NSEG = 64
SEGW = P // NSEG
EPS = 1e-3

CAND = 1280
NW = 32
QW = (B * S) // NW
NCHUNK = P // 16


def _sample_perm(s_num):
    rk = jax.random.key(1)
    rand_num = jnp.abs(jax.random.uniform(rk, (B, P), dtype=jnp.float32))
    sorted_idx = jnp.argsort(rand_num, axis=1)
    return jax.lax.dynamic_slice(sorted_idx, (0, s_num - S), (B, S))


def _tc_body(x_ref, perm_ref, t_ref):
    x5 = x_ref[0]
    x3 = x5[:3, :]
    perm = perm_ref[0, 0]

    iota_p = lax.broadcasted_iota(jnp.int32, (S_BLK, P), 1)

    ohq = (iota_p == perm[:, None]).astype(jnp.float32)
    q = lax.dot_general(ohq, x3, (((1,), (1,)), ((), ())),
                        precision=lax.Precision.HIGHEST)

    xx = jnp.sum(x3 * x3, axis=0)
    xx2 = jnp.sum(q * q, axis=1)
    inner = -2.0 * lax.dot_general(q, x3, (((1,), (0,)), ((), ())))
    z = ((-xx)[None, :] - inner) - xx2[:, None]

    segmax = jnp.max(z.reshape(S_BLK, NSEG, SEGW), axis=2)
    sm = segmax
    for _ in range(K - 1):
        m = jnp.max(sm, axis=1)
        sm = jnp.where(sm == m[:, None], -jnp.inf, sm)
    t_ref[0, 0, :] = jnp.max(sm, axis=1)


def _make_sc_kernel():
    mesh = plsc.VectorSubcoreMesh(core_axis_name="c", subcore_axis_name="s",
                                  num_cores=2, num_subcores=16)

    @functools.partial(
        pl.kernel,
        out_type=jax.ShapeDtypeStruct((B, C, K, S), jnp.float32),
        mesh=mesh,
        compiler_params=pltpu.CompilerParams(needs_layout_passes=False),
        scratch_types=[
            pltpu.VMEM((C, P), jnp.float32),
            pltpu.VMEM((3, P), jnp.float32),
            pltpu.VMEM((P,), jnp.float32),
            pltpu.VMEM((QW,), jnp.int32),
            pltpu.VMEM((QW,), jnp.float32),
            pltpu.VMEM((CAND + 16,), jnp.float32),
            pltpu.VMEM((CAND + 16,), jnp.int32),
            pltpu.VMEM((C, K, QW), jnp.float32),
        ],
    )
    def sc_kernel(x_hbm, perm_hbm, t_hbm, out_hbm,
                  xr, xbr, xxv, pv, tv, dbuf, ibuf, outb):
        cid = lax.axis_index("c")
        sid = lax.axis_index("s")
        w = sid * 2 + cid
        qbase = w * QW
        b = qbase // S
        s0 = qbase % S

        pltpu.sync_copy(x_hbm.at[b], xr)
        pltpu.sync_copy(perm_hbm.at[b, pl.ds(s0, QW)], pv)
        pltpu.sync_copy(t_hbm.at[b, pl.ds(s0, QW)], tv)

        iota16 = lax.broadcasted_iota(jnp.int32, (16,), 0)
        zeros16 = jnp.zeros((16,), jnp.int32)

        def bf16_round(v):
            i = plsc.bitcast(v, jnp.int32)
            r = (i + jnp.int32(0x7FFF) + ((i >> 16) & 1)) & jnp.int32(-65536)
            return plsc.bitcast(r, jnp.float32)

        def xx_loop(ci, _):
            o = ci * 16
            xv = xr[0, pl.ds(o, 16)]
            yv = xr[1, pl.ds(o, 16)]
            zv = xr[2, pl.ds(o, 16)]
            xxv[pl.ds(o, 16)] = (xv * xv + yv * yv) + zv * zv
            xbr[0, pl.ds(o, 16)] = bf16_round(xv)
            xbr[1, pl.ds(o, 16)] = bf16_round(yv)
            xbr[2, pl.ds(o, 16)] = bf16_round(zv)
            return 0
        lax.fori_loop(0, NCHUNK, xx_loop, 0)

        def q_loop(sl, _):
            slv = zeros16 + sl
            p0 = plsc.load_gather(pv, [slv])
            qx = plsc.load_gather(xr, [zeros16, p0])
            qy = plsc.load_gather(xr, [zeros16 + 1, p0])
            qz = plsc.load_gather(xr, [zeros16 + 2, p0])
            xx2 = (qx * qx + qy * qy) + qz * qz
            qxb = bf16_round(qx)
            qyb = bf16_round(qy)
            qzb = bf16_round(qz)
            tth = plsc.load_gather(tv, [slv]) - EPS

            def scan(ci, pos_v):
                o = ci * 16
                xv = xbr[0, pl.ds(o, 16)]
                yv = xbr[1, pl.ds(o, 16)]
                zv = xbr[2, pl.ds(o, 16)]
                xxc = xxv[pl.ds(o, 16)]
                dot = (qxb * xv + qyb * yv) + qzb * zv
                d = ((-xxc) + 2.0 * dot) - xx2
                msk = d >= tth
                tgt = (pos_v + plsc.cumsum(msk.astype(jnp.int32))) - 1
                tgt = jnp.minimum(tgt, jnp.int32(CAND - 1))
                plsc.store_scatter(dbuf, [tgt], d, mask=msk)
                plsc.store_scatter(ibuf, [tgt], iota16 + o, mask=msk)
                return pos_v + plsc.all_reduce_population_count(msk)

            pos_v = lax.fori_loop(0, NCHUNK, scan, zeros16)
            pos = jnp.minimum(jnp.max(pos_v), jnp.int32(CAND))
            nch = (pos + 15) // 16
            last = (nch - 1) * 16
            tail = dbuf[pl.ds(last, 16)]
            dbuf[pl.ds(last, 16)] = jnp.where(iota16 < pos - last, tail,
                                              -jnp.inf)

            a_k, a_v = plsc.sort_key_val(dbuf[pl.ds(0, 16)],
                                         ibuf[pl.ds(0, 16)])

            def merge(j, av):
                ak, avi = av
                o = j * 16
                dk, dv = plsc.sort_key_val(dbuf[pl.ds(o, 16)],
                                           ibuf[pl.ds(o, 16)],
                                           descending=True)
                take = ak >= dk
                uk = jnp.where(take, ak, dk)
                ui = jnp.where(take, avi, dv)
                return tuple(plsc.sort_key_val(uk, ui))

            a_k, a_v = lax.fori_loop(1, nch, merge, (a_k, a_v))
            ridx = lax.rev(a_v, (0,))
            for ci in range(C):
                fv = plsc.load_gather(xr, [zeros16 + ci, ridx])
                plsc.store_scatter(outb, [zeros16 + ci, iota16, slv], fv)
            return 0

        lax.fori_loop(0, QW, q_loop, 0)
        pltpu.sync_copy(outb, out_hbm.at[b, :, :, pl.ds(s0, QW)])

    return sc_kernel


def kernel(x, s_num):
    perm = _sample_perm(s_num).astype(jnp.int32)
    perm3 = perm.reshape(B * NSB, 1, S_BLK)
    t3 = pl.pallas_call(
        _tc_body,
        grid=(B, NSB),
        in_specs=[
            pl.BlockSpec((1, C, P), lambda b, s: (b, 0, 0)),
            pl.BlockSpec((1, 1, S_BLK), lambda b, s: (b * NSB + s, 0, 0)),
        ],
        out_specs=pl.BlockSpec((1, 1, S_BLK), lambda b, s: (b * NSB + s, 0, 0)),
        out_shape=jax.ShapeDtypeStruct((B * NSB, 1, S_BLK), jnp.float32),
    )(x, perm3)
    t = t3.reshape(B, S)
    out = _make_sc_kernel()(x, perm, t)
    return out.reshape(B, C, S, K)


# speedup vs baseline: 7.5773x; 1.8053x over previous
import functools

import jax
import jax.numpy as jnp
from jax import lax
from jax.experimental import pallas as pl
from jax.experimental.pallas import tpu as pltpu
from jax.experimental.pallas import tpu_sc as plsc

B = 4
C = 5
P = 8192
S = 2048
K = 16
S_BLK = 128
NSB = S // S_BLK
NSEG = 64
SEGW = P // NSEG
EPS = 1e-3

CAND = 1280
NW = 32
QW = (B * S) // NW
NCHUNK = P // 16
SEG_GRP = 4


def _sample_perm(s_num):
    rk = jax.random.key(1)
    rand_num = jnp.abs(jax.random.uniform(rk, (B, P), dtype=jnp.float32))
    sorted_idx = jnp.argsort(rand_num, axis=1)
    return jax.lax.dynamic_slice(sorted_idx, (0, s_num - S), (B, S))


def _tc_body(x_ref, perm_ref, segv_ref, segi_ref):
    x5 = x_ref[0]
    x3 = x5[:3, :]
    perm = perm_ref[0, 0]

    iota_p = lax.broadcasted_iota(jnp.int32, (S_BLK, P), 1)

    ohq = (iota_p == perm[:, None]).astype(jnp.float32)
    q = lax.dot_general(ohq, x3, (((1,), (1,)), ((), ())),
                        precision=lax.Precision.HIGHEST)

    xx = jnp.sum(x3 * x3, axis=0, keepdims=True)
    xx2 = jnp.sum(q * q, axis=1)

    lane64 = lax.broadcasted_iota(jnp.int32, (1, NSEG), 1)
    segmax = None
    for j in range(NSEG):
        x3j = x3[:, j * SEGW:(j + 1) * SEGW]
        inner = -2.0 * lax.dot_general(q, x3j, (((1,), (0,)), ((), ())))
        zj = ((-xx[:, j * SEGW:(j + 1) * SEGW]) - inner) - xx2[:, None]
        mj = jnp.max(zj, axis=1)[:, None]
        contrib = mj * (lane64 == j).astype(jnp.float32)
        segmax = contrib if segmax is None else segmax + contrib

    iota_seg = lax.broadcasted_iota(jnp.int32, (S_BLK, NSEG), 1)
    sm = segmax
    vacc = jnp.zeros((S_BLK, NSEG), jnp.float32)
    iacc = jnp.zeros((S_BLK, NSEG), jnp.int32)
    for k in range(NSEG):
        m = jnp.max(sm, axis=1)
        cand = jnp.where(sm == m[:, None], iota_seg, jnp.int32(NSEG))
        j = jnp.min(cand, axis=1)
        ek = (lane64 == k)
        vacc = vacc + m[:, None] * ek.astype(jnp.float32)
        iacc = iacc + j[:, None] * ek.astype(jnp.int32)
        sm = jnp.where(iota_seg == j[:, None], -jnp.inf, sm)
    segv_ref[0] = vacc
    segi_ref[0] = iacc


def _make_sc_kernel():
    mesh = plsc.VectorSubcoreMesh(core_axis_name="c", subcore_axis_name="s",
                                  num_cores=2, num_subcores=16)

    @functools.partial(
        pl.kernel,
        out_type=jax.ShapeDtypeStruct((B, C, K, S), jnp.float32),
        mesh=mesh,
        compiler_params=pltpu.CompilerParams(needs_layout_passes=False),
        scratch_types=[
            pltpu.VMEM((C, P), jnp.float32),
            pltpu.VMEM((P,), jnp.float32),
            pltpu.VMEM((QW,), jnp.int32),
            pltpu.VMEM((QW * NSEG,), jnp.float32),
            pltpu.VMEM((QW * NSEG,), jnp.int32),
            pltpu.VMEM((CAND + 16,), jnp.float32),
            pltpu.VMEM((CAND + 16,), jnp.int32),
            pltpu.VMEM((C, K, QW // 2), jnp.float32),
        ],
    )
    def sc_kernel(x_hbm, perm_hbm, segv_hbm, segi_hbm, out_hbm,
                  xr, xxv, pv, sv, si, dbuf, ibuf, outb):
        cid = lax.axis_index("c")
        sid = lax.axis_index("s")
        w = sid * 2 + cid
        qbase = w * QW
        b = qbase // S
        s0 = qbase % S

        pltpu.sync_copy(x_hbm.at[b], xr)
        pltpu.sync_copy(perm_hbm.at[b, pl.ds(s0, QW)], pv)
        pltpu.sync_copy(segv_hbm.at[b, pl.ds(s0 * NSEG, QW * NSEG)], sv)
        pltpu.sync_copy(segi_hbm.at[b, pl.ds(s0 * NSEG, QW * NSEG)], si)

        iota16 = lax.broadcasted_iota(jnp.int32, (16,), 0)
        zeros16 = jnp.zeros((16,), jnp.int32)

        def bf16_round(v):
            i = plsc.bitcast(v, jnp.int32)
            r = (i + jnp.int32(0x7FFF) + ((i >> 16) & 1)) & jnp.int32(-65536)
            return plsc.bitcast(r, jnp.float32)

        def xx_loop(ci, _):
            o = ci * 16
            xv = xr[0, pl.ds(o, 16)]
            yv = xr[1, pl.ds(o, 16)]
            zv = xr[2, pl.ds(o, 16)]
            xxv[pl.ds(o, 16)] = (xv * xv + yv * yv) + zv * zv
            return 0
        lax.fori_loop(0, NCHUNK, xx_loop, 0)

        def q_loop(sl, _):
            slv = zeros16 + sl
            p0 = plsc.load_gather(pv, [slv])
            qx = plsc.load_gather(xr, [zeros16, p0])
            qy = plsc.load_gather(xr, [zeros16 + 1, p0])
            qz = plsc.load_gather(xr, [zeros16 + 2, p0])
            xx2 = (qx * qx + qy * qy) + qz * qz
            qxb = bf16_round(qx)
            qyb = bf16_round(qy)
            qzb = bf16_round(qz)
            svbase = slv * NSEG
            tth = plsc.load_gather(sv, [svbase + (K - 1)]) - EPS

            def seg_scan(base, pos_v):
                for cc in range(SEGW // 16):
                    ov = (base + cc * 16) + iota16
                    xv = bf16_round(plsc.load_gather(xr, [zeros16, ov]))
                    yv = bf16_round(plsc.load_gather(xr, [zeros16 + 1, ov]))
                    zv = bf16_round(plsc.load_gather(xr, [zeros16 + 2, ov]))
                    xxc = plsc.load_gather(xxv, [ov])
                    dot = (qxb * xv + qyb * yv) + qzb * zv
                    d = ((-xxc) + 2.0 * dot) - xx2
                    msk = d >= tth
                    tgt = (pos_v + plsc.cumsum(msk.astype(jnp.int32))) - 1
                    tgt = jnp.minimum(tgt, jnp.int32(CAND - 1))
                    plsc.store_scatter(dbuf, [tgt], d, mask=msk)
                    plsc.store_scatter(ibuf, [tgt], ov, mask=msk)
                    pos_v = pos_v + plsc.all_reduce_population_count(msk)
                return pos_v

            def grp_cond(carry):
                g, pos_v, cont = carry
                return jnp.logical_and(g < NSEG // SEG_GRP, cont > 0)

            def grp_body(carry):
                g, pos_v, cont = carry
                for u in range(SEG_GRP):
                    sidv = plsc.load_gather(si, [svbase + g * SEG_GRP + u])
                    pos_v = seg_scan(sidv * SEGW, pos_v)
                vlast = plsc.load_gather(
                    sv, [svbase + (g * SEG_GRP + SEG_GRP - 1)])
                more = plsc.all_reduce_population_count(vlast >= tth)
                return g + 1, pos_v, jnp.max(more)

            _, pos_v, _ = lax.while_loop(
                grp_cond, grp_body, (jnp.int32(0), zeros16, jnp.int32(1)))
            pos = jnp.minimum(jnp.max(pos_v), jnp.int32(CAND))
            nch = (pos + 15) // 16
            last = (nch - 1) * 16
            tail = dbuf[pl.ds(last, 16)]
            dbuf[pl.ds(last, 16)] = jnp.where(iota16 < pos - last, tail,
                                              -jnp.inf)

            a_k, a_v = plsc.sort_key_val(dbuf[pl.ds(0, 16)],
                                         ibuf[pl.ds(0, 16)])

            def merge(j, av):
                ak, avi = av
                o = j * 16
                dk, dv = plsc.sort_key_val(dbuf[pl.ds(o, 16)],
                                           ibuf[pl.ds(o, 16)],
                                           descending=True)
                take = ak >= dk
                uk = jnp.where(take, ak, dk)
                ui = jnp.where(take, avi, dv)
                return tuple(plsc.sort_key_val(uk, ui))

            a_k, a_v = lax.fori_loop(1, nch, merge, (a_k, a_v))
            ridx = lax.rev(a_v, (0,))
            slh = zeros16 + (sl & (QW // 2 - 1))
            for ci in range(C):
                fv = plsc.load_gather(xr, [zeros16 + ci, ridx])
                plsc.store_scatter(outb, [zeros16 + ci, iota16, slh], fv)

            @pl.when(sl == QW // 2 - 1)
            def _():
                pltpu.sync_copy(outb, out_hbm.at[b, :, :, pl.ds(s0, QW // 2)])

            @pl.when(sl == QW - 1)
            def _():
                pltpu.sync_copy(outb,
                                out_hbm.at[b, :, :,
                                           pl.ds(s0 + QW // 2, QW // 2)])
            return 0

        lax.fori_loop(0, QW, q_loop, 0)

    return sc_kernel


def kernel(x, s_num):
    perm = _sample_perm(s_num).astype(jnp.int32)
    perm3 = perm.reshape(B * NSB, 1, S_BLK)
    segv3, segi3 = pl.pallas_call(
        _tc_body,
        grid=(B, NSB),
        in_specs=[
            pl.BlockSpec((1, C, P), lambda b, s: (b, 0, 0)),
            pl.BlockSpec((1, 1, S_BLK), lambda b, s: (b * NSB + s, 0, 0)),
        ],
        out_specs=[
            pl.BlockSpec((1, S_BLK, NSEG), lambda b, s: (b * NSB + s, 0, 0)),
            pl.BlockSpec((1, S_BLK, NSEG), lambda b, s: (b * NSB + s, 0, 0)),
        ],
        out_shape=[
            jax.ShapeDtypeStruct((B * NSB, S_BLK, NSEG), jnp.float32),
            jax.ShapeDtypeStruct((B * NSB, S_BLK, NSEG), jnp.int32),
        ],
    )(x, perm3)
    segv = segv3.reshape(B, S * NSEG)
    segi = segi3.reshape(B, S * NSEG)
    out = _make_sc_kernel()(x, perm, segv, segi)
    return out.reshape(B, C, S, K)


# speedup vs baseline: 10.4948x; 1.3850x over previous
import functools

import jax
import jax.numpy as jnp
from jax import lax
from jax.experimental import pallas as pl
from jax.experimental.pallas import tpu as pltpu
from jax.experimental.pallas import tpu_sc as plsc

B = 4
C = 5
P = 8192
S = 2048
K = 16
S_BLK = 128
NSB = S // S_BLK
NSEG = 64
SEGW = P // NSEG
NLIST = 32
EPS = 1e-3

CAND = 1280
NW = 32
QW = (B * S) // NW
NCHUNK = P // 16
SEG_GRP = 4


def _sample_perm(s_num):
    rk = jax.random.key(1)
    rand_num = jnp.abs(jax.random.uniform(rk, (B, P), dtype=jnp.float32))
    sorted_idx = jnp.argsort(rand_num, axis=1)
    return jax.lax.dynamic_slice(sorted_idx, (0, s_num - S), (B, S))


def _tc_body(x_ref, perm_ref, segv_ref, segi_ref):
    x5 = x_ref[0]
    x3 = x5[:3, :]
    perm = perm_ref[0, 0]

    iota_lo = lax.broadcasted_iota(jnp.int32, (S_BLK, SEGW), 1)
    iota_hi = lax.broadcasted_iota(jnp.int32, (S_BLK, NSEG), 1)
    oh_lo = (iota_lo == (perm % SEGW)[:, None]).astype(jnp.float32)
    oh_hi = (iota_hi == (perm // SEGW)[:, None]).astype(jnp.float32)
    x3r = x3.reshape(3, NSEG, SEGW)
    qcols = []
    for c in range(3):
        tc = lax.dot_general(oh_lo, x3r[c], (((1,), (1,)), ((), ())),
                             precision=lax.Precision.HIGHEST)
        qcols.append(jnp.sum(tc * oh_hi, axis=1))
    q = jnp.stack(qcols, axis=1)

    xx = jnp.sum(x3 * x3, axis=0, keepdims=True)
    xx2 = jnp.sum(q * q, axis=1)

    lane64 = lax.broadcasted_iota(jnp.int32, (1, NSEG), 1)
    segmax = None
    for j in range(NSEG):
        x3j = x3[:, j * SEGW:(j + 1) * SEGW]
        inner = -2.0 * lax.dot_general(q, x3j, (((1,), (0,)), ((), ())))
        zj = ((-xx[:, j * SEGW:(j + 1) * SEGW]) - inner) - xx2[:, None]
        mj = jnp.max(zj, axis=1)[:, None]
        contrib = mj * (lane64 == j).astype(jnp.float32)
        segmax = contrib if segmax is None else segmax + contrib

    iota_seg = lax.broadcasted_iota(jnp.int32, (S_BLK, NSEG), 1)
    lane32 = lax.broadcasted_iota(jnp.int32, (1, NLIST), 1)
    sm = segmax
    vacc = jnp.zeros((S_BLK, NLIST), jnp.float32)
    iacc = jnp.zeros((S_BLK, NLIST), jnp.int32)
    for k in range(NLIST):
        m = jnp.max(sm, axis=1)
        cand = jnp.where(sm == m[:, None], iota_seg, jnp.int32(NSEG))
        j = jnp.min(cand, axis=1)
        ek = (lane32 == k)
        vacc = vacc + m[:, None] * ek.astype(jnp.float32)
        iacc = iacc + j[:, None] * ek.astype(jnp.int32)
        sm = jnp.where(iota_seg == j[:, None], -jnp.inf, sm)
    segv_ref[0] = vacc
    segi_ref[0] = iacc


def _make_sc_kernel():
    mesh = plsc.VectorSubcoreMesh(core_axis_name="c", subcore_axis_name="s",
                                  num_cores=2, num_subcores=16)

    @functools.partial(
        pl.kernel,
        out_type=jax.ShapeDtypeStruct((B, C, K, S), jnp.float32),
        mesh=mesh,
        compiler_params=pltpu.CompilerParams(needs_layout_passes=False),
        scratch_types=[
            pltpu.VMEM((C, P), jnp.float32),
            pltpu.VMEM((P,), jnp.float32),
            pltpu.VMEM((QW,), jnp.int32),
            pltpu.VMEM((QW * NLIST,), jnp.float32),
            pltpu.VMEM((QW * NLIST,), jnp.int32),
            pltpu.VMEM((CAND + 16,), jnp.float32),
            pltpu.VMEM((CAND + 16,), jnp.int32),
            pltpu.VMEM((C, K, QW // 2), jnp.float32),
        ],
    )
    def sc_kernel(x_hbm, perm_hbm, segv_hbm, segi_hbm, out_hbm,
                  xr, xxv, pv, sv, si, dbuf, ibuf, outb):
        cid = lax.axis_index("c")
        sid = lax.axis_index("s")
        w = sid * 2 + cid
        qbase = w * QW
        b = qbase // S
        s0 = qbase % S

        pltpu.sync_copy(x_hbm.at[b], xr)
        pltpu.sync_copy(perm_hbm.at[b, pl.ds(s0, QW)], pv)
        pltpu.sync_copy(segv_hbm.at[b, pl.ds(s0 * NLIST, QW * NLIST)], sv)
        pltpu.sync_copy(segi_hbm.at[b, pl.ds(s0 * NLIST, QW * NLIST)], si)

        iota16 = lax.broadcasted_iota(jnp.int32, (16,), 0)
        zeros16 = jnp.zeros((16,), jnp.int32)

        def bf16_round(v):
            i = plsc.bitcast(v, jnp.int32)
            r = (i + jnp.int32(0x7FFF) + ((i >> 16) & 1)) & jnp.int32(-65536)
            return plsc.bitcast(r, jnp.float32)

        def xx_loop(ci, _):
            o = ci * 16
            xv = xr[0, pl.ds(o, 16)]
            yv = xr[1, pl.ds(o, 16)]
            zv = xr[2, pl.ds(o, 16)]
            xxv[pl.ds(o, 16)] = (xv * xv + yv * yv) + zv * zv
            return 0
        lax.fori_loop(0, NCHUNK, xx_loop, 0)

        def q_loop(sl, _):
            slv = zeros16 + sl
            p0 = plsc.load_gather(pv, [slv])
            qx = plsc.load_gather(xr, [zeros16, p0])
            qy = plsc.load_gather(xr, [zeros16 + 1, p0])
            qz = plsc.load_gather(xr, [zeros16 + 2, p0])
            xx2 = (qx * qx + qy * qy) + qz * qz
            qxb = bf16_round(qx)
            qyb = bf16_round(qy)
            qzb = bf16_round(qz)
            svbase = slv * NLIST
            tth = plsc.load_gather(sv, [svbase + (K - 1)]) - EPS

            def seg_scan(base, pos_v):
                for cc in range(SEGW // 16):
                    ov = (base + cc * 16) + iota16
                    xv = bf16_round(plsc.load_gather(xr, [zeros16, ov]))
                    yv = bf16_round(plsc.load_gather(xr, [zeros16 + 1, ov]))
                    zv = bf16_round(plsc.load_gather(xr, [zeros16 + 2, ov]))
                    xxc = plsc.load_gather(xxv, [ov])
                    dot = (qxb * xv + qyb * yv) + qzb * zv
                    d = ((-xxc) + 2.0 * dot) - xx2
                    msk = d >= tth
                    tgt = (pos_v + plsc.cumsum(msk.astype(jnp.int32))) - 1
                    tgt = jnp.minimum(tgt, jnp.int32(CAND - 1))
                    plsc.store_scatter(dbuf, [tgt], d, mask=msk)
                    plsc.store_scatter(ibuf, [tgt], ov, mask=msk)
                    pos_v = pos_v + plsc.all_reduce_population_count(msk)
                return pos_v

            def grp_cond(carry):
                g, pos_v, cont = carry
                return jnp.logical_and(g < NLIST // SEG_GRP, cont > 0)

            def grp_body(carry):
                g, pos_v, cont = carry
                for u in range(SEG_GRP):
                    sidv = plsc.load_gather(si, [svbase + g * SEG_GRP + u])
                    pos_v = seg_scan(sidv * SEGW, pos_v)
                vlast = plsc.load_gather(
                    sv, [svbase + (g * SEG_GRP + SEG_GRP - 1)])
                more = plsc.all_reduce_population_count(vlast >= tth)
                return g + 1, pos_v, jnp.max(more)

            _, pos_v, cont_end = lax.while_loop(
                grp_cond, grp_body, (jnp.int32(0), zeros16, jnp.int32(1)))

            def full_rescan(_):
                def body(j, pv2):
                    return seg_scan((zeros16 + j) * SEGW, pv2)
                return lax.fori_loop(0, NSEG, body, zeros16)

            pos_v = lax.cond(cont_end > 0, full_rescan, lambda _: pos_v, 0)
            pos = jnp.minimum(jnp.max(pos_v), jnp.int32(CAND))
            nch = (pos + 15) // 16
            last = (nch - 1) * 16
            tail = dbuf[pl.ds(last, 16)]
            dbuf[pl.ds(last, 16)] = jnp.where(iota16 < pos - last, tail,
                                              -jnp.inf)

            a_k, a_v = plsc.sort_key_val(dbuf[pl.ds(0, 16)],
                                         ibuf[pl.ds(0, 16)])

            def merge(j, av):
                ak, avi = av
                o = j * 16
                dk, dv = plsc.sort_key_val(dbuf[pl.ds(o, 16)],
                                           ibuf[pl.ds(o, 16)],
                                           descending=True)
                take = ak >= dk
                uk = jnp.where(take, ak, dk)
                ui = jnp.where(take, avi, dv)
                return tuple(plsc.sort_key_val(uk, ui))

            a_k, a_v = lax.fori_loop(1, nch, merge, (a_k, a_v))
            ridx = lax.rev(a_v, (0,))
            slh = zeros16 + (sl & (QW // 2 - 1))
            for ci in range(C):
                fv = plsc.load_gather(xr, [zeros16 + ci, ridx])
                plsc.store_scatter(outb, [zeros16 + ci, iota16, slh], fv)

            @pl.when(sl == QW // 2 - 1)
            def _():
                pltpu.sync_copy(outb, out_hbm.at[b, :, :, pl.ds(s0, QW // 2)])

            @pl.when(sl == QW - 1)
            def _():
                pltpu.sync_copy(outb,
                                out_hbm.at[b, :, :,
                                           pl.ds(s0 + QW // 2, QW // 2)])
            return 0

        lax.fori_loop(0, QW, q_loop, 0)

    return sc_kernel


def kernel(x, s_num):
    perm = _sample_perm(s_num).astype(jnp.int32)
    perm3 = perm.reshape(B * NSB, 1, S_BLK)
    segv3, segi3 = pl.pallas_call(
        _tc_body,
        grid=(B, NSB),
        in_specs=[
            pl.BlockSpec((1, C, P), lambda b, s: (b, 0, 0)),
            pl.BlockSpec((1, 1, S_BLK), lambda b, s: (b * NSB + s, 0, 0)),
        ],
        out_specs=[
            pl.BlockSpec((1, S_BLK, NLIST), lambda b, s: (b * NSB + s, 0, 0)),
            pl.BlockSpec((1, S_BLK, NLIST), lambda b, s: (b * NSB + s, 0, 0)),
        ],
        out_shape=[
            jax.ShapeDtypeStruct((B * NSB, S_BLK, NLIST), jnp.float32),
            jax.ShapeDtypeStruct((B * NSB, S_BLK, NLIST), jnp.int32),
        ],
    )(x, perm3)
    segv = segv3.reshape(B, S * NLIST)
    segi = segi3.reshape(B, S * NLIST)
    out = _make_sc_kernel()(x, perm, segv, segi)
    return out.reshape(B, C, S, K)


# speedup vs baseline: 11.7219x; 1.1169x over previous
import functools

import jax
import jax.numpy as jnp
from jax import lax
from jax.experimental import pallas as pl
from jax.experimental.pallas import tpu as pltpu
from jax.experimental.pallas import tpu_sc as plsc

B = 4
C = 5
P = 8192
S = 2048
K = 16
S_BLK = 128
NSB = S // S_BLK
NSEG = 64
SEGW = P // NSEG
NLIST = 24
EPS = 1e-3

CAND = 1280
NW = 32
QW = (B * S) // NW
NCHUNK = P // 16
SEG_GRP = 2


def _sample_perm(s_num):
    rk = jax.random.key(1)
    rand_num = jnp.abs(jax.random.uniform(rk, (B, P), dtype=jnp.float32))
    sorted_idx = jnp.argsort(rand_num, axis=1)
    return jax.lax.dynamic_slice(sorted_idx, (0, s_num - S), (B, S))


def _tc_body(x_ref, perm_ref, segv_ref, segi_ref):
    x5 = x_ref[0]
    x3 = x5[:3, :]
    perm = perm_ref[0, 0]

    iota_lo = lax.broadcasted_iota(jnp.int32, (S_BLK, SEGW), 1)
    iota_hi = lax.broadcasted_iota(jnp.int32, (S_BLK, NSEG), 1)
    oh_lo = (iota_lo == (perm % SEGW)[:, None]).astype(jnp.float32)
    oh_hi = (iota_hi == (perm // SEGW)[:, None]).astype(jnp.float32)
    x3r = x3.reshape(3, NSEG, SEGW)
    qcols = []
    for c in range(3):
        tc = lax.dot_general(oh_lo, x3r[c], (((1,), (1,)), ((), ())),
                             precision=lax.Precision.HIGHEST)
        qcols.append(jnp.sum(tc * oh_hi, axis=1))
    q = jnp.stack(qcols, axis=1)

    xx = jnp.sum(x3 * x3, axis=0, keepdims=True)
    xx2 = jnp.sum(q * q, axis=1)

    lane64 = lax.broadcasted_iota(jnp.int32, (1, NSEG), 1)
    segmax = None
    for j in range(NSEG):
        x3j = x3[:, j * SEGW:(j + 1) * SEGW]
        inner = -2.0 * lax.dot_general(q, x3j, (((1,), (0,)), ((), ())))
        zj = ((-xx[:, j * SEGW:(j + 1) * SEGW]) - inner) - xx2[:, None]
        mj = jnp.max(zj, axis=1)[:, None]
        contrib = mj * (lane64 == j).astype(jnp.float32)
        segmax = contrib if segmax is None else segmax + contrib

    iota_seg = lax.broadcasted_iota(jnp.int32, (S_BLK, NSEG), 1)
    lane32 = lax.broadcasted_iota(jnp.int32, (1, NLIST), 1)
    sm = segmax
    vacc = jnp.zeros((S_BLK, NLIST), jnp.float32)
    iacc = jnp.zeros((S_BLK, NLIST), jnp.int32)
    for k in range(NLIST):
        m = jnp.max(sm, axis=1)
        cand = jnp.where(sm == m[:, None], iota_seg, jnp.int32(NSEG))
        j = jnp.min(cand, axis=1)
        ek = (lane32 == k)
        vacc = vacc + m[:, None] * ek.astype(jnp.float32)
        iacc = iacc + j[:, None] * ek.astype(jnp.int32)
        sm = jnp.where(iota_seg == j[:, None], -jnp.inf, sm)
    segv_ref[0] = vacc
    segi_ref[0] = iacc


def _make_sc_kernel():
    mesh = plsc.VectorSubcoreMesh(core_axis_name="c", subcore_axis_name="s",
                                  num_cores=2, num_subcores=16)

    @functools.partial(
        pl.kernel,
        out_type=jax.ShapeDtypeStruct((B, C, K, S), jnp.float32),
        mesh=mesh,
        compiler_params=pltpu.CompilerParams(needs_layout_passes=False),
        scratch_types=[
            pltpu.VMEM((C, P), jnp.float32),
            pltpu.VMEM((P,), jnp.float32),
            pltpu.VMEM((QW,), jnp.int32),
            pltpu.VMEM((QW * NLIST,), jnp.float32),
            pltpu.VMEM((QW * NLIST,), jnp.int32),
            pltpu.VMEM((CAND + 16,), jnp.float32),
            pltpu.VMEM((CAND + 16,), jnp.int32),
            pltpu.VMEM((C, K, QW // 2), jnp.float32),
        ],
    )
    def sc_kernel(x_hbm, perm_hbm, segv_hbm, segi_hbm, out_hbm,
                  xr, xxv, pv, sv, si, dbuf, ibuf, outb):
        cid = lax.axis_index("c")
        sid = lax.axis_index("s")
        w = sid * 2 + cid
        qbase = w * QW
        b = qbase // S
        s0 = qbase % S

        pltpu.sync_copy(x_hbm.at[b], xr)
        pltpu.sync_copy(perm_hbm.at[b, pl.ds(s0, QW)], pv)
        pltpu.sync_copy(segv_hbm.at[b, pl.ds(s0 * NLIST, QW * NLIST)], sv)
        pltpu.sync_copy(segi_hbm.at[b, pl.ds(s0 * NLIST, QW * NLIST)], si)

        iota16 = lax.broadcasted_iota(jnp.int32, (16,), 0)
        zeros16 = jnp.zeros((16,), jnp.int32)

        def bf16_round(v):
            i = plsc.bitcast(v, jnp.int32)
            r = (i + jnp.int32(0x7FFF) + ((i >> 16) & 1)) & jnp.int32(-65536)
            return plsc.bitcast(r, jnp.float32)

        def xx_loop(ci, _):
            o = ci * 16
            xv = xr[0, pl.ds(o, 16)]
            yv = xr[1, pl.ds(o, 16)]
            zv = xr[2, pl.ds(o, 16)]
            xxv[pl.ds(o, 16)] = (xv * xv + yv * yv) + zv * zv
            return 0
        lax.fori_loop(0, NCHUNK, xx_loop, 0)

        def q_loop(sl, _):
            slv = zeros16 + sl
            p0 = plsc.load_gather(pv, [slv])
            qx = plsc.load_gather(xr, [zeros16, p0])
            qy = plsc.load_gather(xr, [zeros16 + 1, p0])
            qz = plsc.load_gather(xr, [zeros16 + 2, p0])
            xx2 = (qx * qx + qy * qy) + qz * qz
            qxb = bf16_round(qx)
            qyb = bf16_round(qy)
            qzb = bf16_round(qz)
            svbase = slv * NLIST
            tth = plsc.load_gather(sv, [svbase + (K - 1)]) - EPS

            def seg_scan(base, pos_v):
                for cc in range(SEGW // 16):
                    ov = (base + cc * 16) + iota16
                    xv = bf16_round(plsc.load_gather(xr, [zeros16, ov]))
                    yv = bf16_round(plsc.load_gather(xr, [zeros16 + 1, ov]))
                    zv = bf16_round(plsc.load_gather(xr, [zeros16 + 2, ov]))
                    xxc = plsc.load_gather(xxv, [ov])
                    dot = (qxb * xv + qyb * yv) + qzb * zv
                    d = ((-xxc) + 2.0 * dot) - xx2
                    msk = d >= tth
                    tgt = (pos_v + plsc.cumsum(msk.astype(jnp.int32))) - 1
                    tgt = jnp.minimum(tgt, jnp.int32(CAND - 1))
                    plsc.store_scatter(dbuf, [tgt], d, mask=msk)
                    plsc.store_scatter(ibuf, [tgt], ov, mask=msk)
                    pos_v = pos_v + plsc.all_reduce_population_count(msk)
                return pos_v

            def grp_cond(carry):
                g, pos_v, cont = carry
                return jnp.logical_and(g < NLIST // SEG_GRP, cont > 0)

            def grp_body(carry):
                g, pos_v, cont = carry
                for u in range(SEG_GRP):
                    sidv = plsc.load_gather(si, [svbase + g * SEG_GRP + u])
                    pos_v = seg_scan(sidv * SEGW, pos_v)
                vlast = plsc.load_gather(
                    sv, [svbase + (g * SEG_GRP + SEG_GRP - 1)])
                more = plsc.all_reduce_population_count(vlast >= tth)
                return g + 1, pos_v, jnp.max(more)

            _, pos_v, cont_end = lax.while_loop(
                grp_cond, grp_body, (jnp.int32(0), zeros16, jnp.int32(1)))

            def full_rescan(_):
                def body(j, pv2):
                    return seg_scan((zeros16 + j) * SEGW, pv2)
                return lax.fori_loop(0, NSEG, body, zeros16)

            pos_v = lax.cond(cont_end > 0, full_rescan, lambda _: pos_v, 0)
            pos = jnp.minimum(jnp.max(pos_v), jnp.int32(CAND))
            nch = (pos + 15) // 16
            last = (nch - 1) * 16
            tail = dbuf[pl.ds(last, 16)]
            dbuf[pl.ds(last, 16)] = jnp.where(iota16 < pos - last, tail,
                                              -jnp.inf)

            a_k, a_v = plsc.sort_key_val(dbuf[pl.ds(0, 16)],
                                         ibuf[pl.ds(0, 16)])

            def merge(j, av):
                ak, avi = av
                o = j * 16
                dk, dv = plsc.sort_key_val(dbuf[pl.ds(o, 16)],
                                           ibuf[pl.ds(o, 16)],
                                           descending=True)
                take = ak >= dk
                uk = jnp.where(take, ak, dk)
                ui = jnp.where(take, avi, dv)
                return tuple(plsc.sort_key_val(uk, ui))

            a_k, a_v = lax.fori_loop(1, nch, merge, (a_k, a_v))
            ridx = lax.rev(a_v, (0,))
            slh = zeros16 + (sl & (QW // 2 - 1))
            for ci in range(C):
                fv = plsc.load_gather(xr, [zeros16 + ci, ridx])
                plsc.store_scatter(outb, [zeros16 + ci, iota16, slh], fv)

            @pl.when(sl == QW // 2 - 1)
            def _():
                pltpu.sync_copy(outb, out_hbm.at[b, :, :, pl.ds(s0, QW // 2)])

            @pl.when(sl == QW - 1)
            def _():
                pltpu.sync_copy(outb,
                                out_hbm.at[b, :, :,
                                           pl.ds(s0 + QW // 2, QW // 2)])
            return 0

        lax.fori_loop(0, QW, q_loop, 0)

    return sc_kernel


def kernel(x, s_num):
    perm = _sample_perm(s_num).astype(jnp.int32)
    perm3 = perm.reshape(B * NSB, 1, S_BLK)
    segv3, segi3 = pl.pallas_call(
        _tc_body,
        grid=(B, NSB),
        in_specs=[
            pl.BlockSpec((1, C, P), lambda b, s: (b, 0, 0)),
            pl.BlockSpec((1, 1, S_BLK), lambda b, s: (b * NSB + s, 0, 0)),
        ],
        out_specs=[
            pl.BlockSpec((1, S_BLK, NLIST), lambda b, s: (b * NSB + s, 0, 0)),
            pl.BlockSpec((1, S_BLK, NLIST), lambda b, s: (b * NSB + s, 0, 0)),
        ],
        out_shape=[
            jax.ShapeDtypeStruct((B * NSB, S_BLK, NLIST), jnp.float32),
            jax.ShapeDtypeStruct((B * NSB, S_BLK, NLIST), jnp.int32),
        ],
    )(x, perm3)
    segv = segv3.reshape(B, S * NLIST)
    segi = segi3.reshape(B, S * NLIST)
    out = _make_sc_kernel()(x, perm, segv, segi)
    return out.reshape(B, C, S, K)
